# Initial kernel scaffold; baseline (speedup 1.0000x reference)
import jax
import jax.numpy as jnp
from jax.experimental import pallas as pl


def kernel(h, r, t, ent_embedding, rel_hyperplane_embedding, rel_norm_embedding):
    raise NotImplementedError("write your pallas kernel here")


# speedup vs baseline: 3.6644x; 3.6644x over previous
import functools

import jax
import jax.numpy as jnp
from jax import lax
from jax.experimental import pallas as pl
from jax.experimental.pallas import tpu as pltpu
from jax.experimental.pallas import tpu_sc as plsc

D = 128
B = 16384
NC, NS, L = 2, 16, 16
NW = NC * NS
RPW = B // NW
CH = 64
NCHUNK = RPW // CH
NBUF = 2


def _rsqrt_nr(x):
    xi = plsc.bitcast(x, jnp.int32)
    yi = jnp.int32(0x5F3759DF) - (xi >> 1)
    y = plsc.bitcast(yi, jnp.float32)
    for _ in range(3):
        y = y * (1.5 - (0.5 * x) * y * y)
    return y


def _make_kernel():
    mesh = plsc.VectorSubcoreMesh(core_axis_name="c", subcore_axis_name="s")

    @functools.partial(
        pl.kernel,
        mesh=mesh,
        out_type=jax.ShapeDtypeStruct((B,), jnp.float32),
        compiler_params=pltpu.CompilerParams(needs_layout_passes=False),
        scratch_types=[
            pltpu.VMEM((RPW,), jnp.int32),
            pltpu.VMEM((RPW,), jnp.int32),
            pltpu.VMEM((RPW,), jnp.int32),
            pltpu.VMEM((NBUF, CH, D), jnp.float32),
            pltpu.VMEM((NBUF, CH, D), jnp.float32),
            pltpu.VMEM((NBUF, CH, D), jnp.float32),
            pltpu.VMEM((NBUF, CH, D), jnp.float32),
            pltpu.VMEM((RPW,), jnp.float32),
            pltpu.VMEM((RPW,), jnp.float32),
            pltpu.SemaphoreType.DMA,
            pltpu.SemaphoreType.DMA,
        ],
    )
    def vec2tail(h_hbm, r_hbm, t_hbm, ent_hbm, hyp_hbm, nrm_hbm, out_hbm,
                 hidx, tidx, ridx, hbuf, tbuf, dbuf, nbuf, sbuf, obuf,
                 sem0, sem1):
        wid = lax.axis_index("s") * NC + lax.axis_index("c")
        base = wid * RPW
        pltpu.sync_copy(h_hbm.at[pl.ds(base, RPW)], hidx)
        pltpu.sync_copy(t_hbm.at[pl.ds(base, RPW)], tidx)
        pltpu.sync_copy(r_hbm.at[pl.ds(base, RPW)], ridx)

        sems = (sem0, sem1)
        lane15 = lax.iota(jnp.int32, L) == (L - 1)

        def fire(g):
            slot = g % NBUF
            sem = sems[slot]
            sl = pl.ds(g * CH, CH)
            return [
                pltpu.async_copy(ent_hbm.at[hidx.at[sl]], hbuf.at[slot], sem),
                pltpu.async_copy(ent_hbm.at[tidx.at[sl]], tbuf.at[slot], sem),
                pltpu.async_copy(hyp_hbm.at[ridx.at[sl]], dbuf.at[slot], sem),
                pltpu.async_copy(nrm_hbm.at[ridx.at[sl]], nbuf.at[slot], sem),
            ]

        pending = {0: fire(0)}
        for g in range(NCHUNK):
            if g + 1 < NCHUNK:
                pending[g + 1] = fire(g + 1)
            for cp in pending.pop(g):
                cp.wait()
            slot = g % NBUF

            def row(i, _, slot=slot, g=g):
                accq = accp = accn = accb = None
                for cb in range(D // L):
                    sl = pl.ds(cb * L, L)
                    hv = hbuf[slot, i, sl]
                    tv = tbuf[slot, i, sl]
                    dv = dbuf[slot, i, sl]
                    nv = nbuf[slot, i, sl]
                    u = hv - tv + dv
                    if cb == 0:
                        accq, accp = u * u, u * nv
                        accn, accb = nv * nv, dv * nv
                    else:
                        accq = accq + u * u
                        accp = accp + u * nv
                        accn = accn + nv * nv
                        accb = accb + dv * nv
                q2 = plsc.cumsum(accq)
                pn = plsc.cumsum(accp)
                n2 = plsc.cumsum(accn)
                bb = plsc.cumsum(accb)
                c = (pn - bb) / jnp.maximum(n2, 1e-24)
                s = jnp.maximum(q2 - (2.0 * c) * pn + (c * c) * n2, 0.0)
                plsc.store_scatter(
                    sbuf, [jnp.broadcast_to(g * CH + i, (L,))], s, mask=lane15)
                return 0

            lax.fori_loop(0, CH, row, 0)

        for k in range(RPW // L):
            sl = pl.ds(k * L, L)
            x = sbuf[sl]
            obuf[sl] = x * _rsqrt_nr(x)
        pltpu.sync_copy(obuf, out_hbm.at[pl.ds(base, RPW)])

    return vec2tail


_vec2tail = _make_kernel()


def kernel(h, r, t, ent_embedding, rel_hyperplane_embedding,
           rel_norm_embedding):
    return _vec2tail(h.astype(jnp.int32), r.astype(jnp.int32),
                     t.astype(jnp.int32), ent_embedding,
                     rel_hyperplane_embedding, rel_norm_embedding)


# speedup vs baseline: 3.9282x; 1.0720x over previous
import functools

import jax
import jax.numpy as jnp
from jax import lax
from jax.experimental import pallas as pl
from jax.experimental.pallas import tpu as pltpu
from jax.experimental.pallas import tpu_sc as plsc

D = 128
B = 16384
NC, NS, L = 2, 16, 16
NW = NC * NS
RPW = B // NW
CH = 64
NCHUNK = RPW // CH
NBUF = 2


def _rsqrt_nr(x):
    xi = plsc.bitcast(x, jnp.int32)
    yi = jnp.int32(0x5F3759DF) - (xi >> 1)
    y = plsc.bitcast(yi, jnp.float32)
    for _ in range(3):
        y = y * (1.5 - (0.5 * x) * y * y)
    return y


def _make_kernel():
    mesh = plsc.VectorSubcoreMesh(core_axis_name="c", subcore_axis_name="s")

    @functools.partial(
        pl.kernel,
        mesh=mesh,
        out_type=jax.ShapeDtypeStruct((B,), jnp.float32),
        compiler_params=pltpu.CompilerParams(needs_layout_passes=False),
        scratch_types=[
            pltpu.VMEM((RPW,), jnp.int32),
            pltpu.VMEM((RPW,), jnp.int32),
            pltpu.VMEM((RPW,), jnp.int32),
            pltpu.VMEM((NBUF, CH, D), jnp.float32),
            pltpu.VMEM((NBUF, CH, D), jnp.float32),
            pltpu.VMEM((NBUF, CH, D), jnp.float32),
            pltpu.VMEM((NBUF, CH, D), jnp.float32),
            pltpu.VMEM((RPW,), jnp.float32),
            pltpu.VMEM((RPW,), jnp.float32),
            pltpu.SemaphoreType.DMA,
            pltpu.SemaphoreType.DMA,
        ],
    )
    def vec2tail(h_hbm, r_hbm, t_hbm, ent_hbm, hyp_hbm, nrm_hbm, out_hbm,
                 hidx, tidx, ridx, hbuf, tbuf, dbuf, nbuf, sbuf, obuf,
                 sem0, sem1):
        wid = lax.axis_index("s") * NC + lax.axis_index("c")
        base = wid * RPW
        pltpu.sync_copy(h_hbm.at[pl.ds(base, RPW)], hidx)
        pltpu.sync_copy(t_hbm.at[pl.ds(base, RPW)], tidx)
        pltpu.sync_copy(r_hbm.at[pl.ds(base, RPW)], ridx)

        sems = (sem0, sem1)
        lane15 = lax.iota(jnp.int32, L) == (L - 1)

        def fire(g):
            slot = g % NBUF
            sem = sems[slot]
            sl = pl.ds(g * CH, CH)
            return [
                pltpu.async_copy(ent_hbm.at[hidx.at[sl]], hbuf.at[slot], sem),
                pltpu.async_copy(ent_hbm.at[tidx.at[sl]], tbuf.at[slot], sem),
                pltpu.async_copy(hyp_hbm.at[ridx.at[sl]], dbuf.at[slot], sem),
                pltpu.async_copy(nrm_hbm.at[ridx.at[sl]], nbuf.at[slot], sem),
            ]

        pending = {0: fire(0)}
        for g in range(NCHUNK):
            if g + 1 < NCHUNK:
                pending[g + 1] = fire(g + 1)
            for cp in pending.pop(g):
                cp.wait()
            slot = g % NBUF

            @plsc.parallel_loop(0, CH, 1, unroll=4)
            def row(i, slot=slot, g=g):
                accq = accp = accn = accb = None
                for cb in range(D // L):
                    sl = pl.ds(cb * L, L)
                    hv = hbuf[slot, i, sl]
                    tv = tbuf[slot, i, sl]
                    dv = dbuf[slot, i, sl]
                    nv = nbuf[slot, i, sl]
                    u = hv - tv + dv
                    if cb == 0:
                        accq, accp = u * u, u * nv
                        accn, accb = nv * nv, dv * nv
                    else:
                        accq = accq + u * u
                        accp = accp + u * nv
                        accn = accn + nv * nv
                        accb = accb + dv * nv
                q2 = plsc.cumsum(accq)
                pn = plsc.cumsum(accp)
                n2 = plsc.cumsum(accn)
                bb = plsc.cumsum(accb)
                c = (pn - bb) / jnp.maximum(n2, 1e-24)
                s = jnp.maximum(q2 - (2.0 * c) * pn + (c * c) * n2, 0.0)
                plsc.store_scatter(
                    sbuf, [jnp.broadcast_to(g * CH + i, (L,))], s, mask=lane15)

        for k in range(RPW // L):
            sl = pl.ds(k * L, L)
            x = sbuf[sl]
            obuf[sl] = x * _rsqrt_nr(x)
        pltpu.sync_copy(obuf, out_hbm.at[pl.ds(base, RPW)])

    return vec2tail


_vec2tail = _make_kernel()


def kernel(h, r, t, ent_embedding, rel_hyperplane_embedding,
           rel_norm_embedding):
    return _vec2tail(h.astype(jnp.int32), r.astype(jnp.int32),
                     t.astype(jnp.int32), ent_embedding,
                     rel_hyperplane_embedding, rel_norm_embedding)


# speedup vs baseline: 3.9815x; 1.0136x over previous
import functools

import jax
import jax.numpy as jnp
from jax import lax
from jax.experimental import pallas as pl
from jax.experimental.pallas import tpu as pltpu
from jax.experimental.pallas import tpu_sc as plsc

D = 128
B = 16384
NC, NS, L = 2, 16, 16
NW = NC * NS
RPW = B // NW
CH = 64
NCHUNK = RPW // CH
NBUF = 3


def _rsqrt_nr(x):
    xi = plsc.bitcast(x, jnp.int32)
    yi = jnp.int32(0x5F3759DF) - (xi >> 1)
    y = plsc.bitcast(yi, jnp.float32)
    for _ in range(3):
        y = y * (1.5 - (0.5 * x) * y * y)
    return y


def _make_kernel():
    mesh = plsc.VectorSubcoreMesh(core_axis_name="c", subcore_axis_name="s")

    @functools.partial(
        pl.kernel,
        mesh=mesh,
        out_type=jax.ShapeDtypeStruct((B,), jnp.float32),
        compiler_params=pltpu.CompilerParams(needs_layout_passes=False),
        scratch_types=[
            pltpu.VMEM((RPW,), jnp.int32),
            pltpu.VMEM((RPW,), jnp.int32),
            pltpu.VMEM((RPW,), jnp.int32),
            pltpu.VMEM((NBUF, CH, D), jnp.float32),
            pltpu.VMEM((NBUF, CH, D), jnp.float32),
            pltpu.VMEM((NBUF, CH, 2 * D), jnp.float32),
            pltpu.VMEM((RPW,), jnp.float32),
            pltpu.VMEM((RPW,), jnp.float32),
            pltpu.SemaphoreType.DMA,
            pltpu.SemaphoreType.DMA,
            pltpu.SemaphoreType.DMA,
            pltpu.SemaphoreType.DMA,
        ],
    )
    def vec2tail(h_hbm, r_hbm, t_hbm, ent_hbm, rel_hbm, out_hbm,
                 hidx, tidx, ridx, hbuf, tbuf, rbuf, sbuf, obuf,
                 sem0, sem1, sem2, isem):
        wid = lax.axis_index("s") * NC + lax.axis_index("c")
        base = wid * RPW
        icps = [
            pltpu.async_copy(h_hbm.at[pl.ds(base, RPW)], hidx, isem),
            pltpu.async_copy(t_hbm.at[pl.ds(base, RPW)], tidx, isem),
            pltpu.async_copy(r_hbm.at[pl.ds(base, RPW)], ridx, isem),
        ]
        for cp in icps:
            cp.wait()

        sems = (sem0, sem1, sem2)
        lane15 = lax.iota(jnp.int32, L) == (L - 1)

        def fire(g):
            slot = g % NBUF
            sem = sems[slot]
            sl = pl.ds(g * CH, CH)
            return [
                pltpu.async_copy(ent_hbm.at[hidx.at[sl]], hbuf.at[slot], sem),
                pltpu.async_copy(ent_hbm.at[tidx.at[sl]], tbuf.at[slot], sem),
                pltpu.async_copy(rel_hbm.at[ridx.at[sl]], rbuf.at[slot], sem),
            ]

        pending = {g: fire(g) for g in range(min(NBUF - 1, NCHUNK))}
        for g in range(NCHUNK):
            if g + NBUF - 1 < NCHUNK:
                pending[g + NBUF - 1] = fire(g + NBUF - 1)
            for cp in pending.pop(g):
                cp.wait()
            slot = g % NBUF

            @plsc.parallel_loop(0, CH, 1, unroll=4)
            def row(i, slot=slot, g=g):
                accq = accp = accn = accb = None
                for cb in range(D // L):
                    sl = pl.ds(cb * L, L)
                    hv = hbuf[slot, i, sl]
                    tv = tbuf[slot, i, sl]
                    dv = rbuf[slot, i, sl]
                    nv = rbuf[slot, i, pl.ds(D + cb * L, L)]
                    u = hv - tv + dv
                    if cb == 0:
                        accq, accp = u * u, u * nv
                        accn, accb = nv * nv, dv * nv
                    else:
                        accq = accq + u * u
                        accp = accp + u * nv
                        accn = accn + nv * nv
                        accb = accb + dv * nv
                q2 = plsc.cumsum(accq)
                pn = plsc.cumsum(accp)
                n2 = plsc.cumsum(accn)
                bb = plsc.cumsum(accb)
                c = (pn - bb) / jnp.maximum(n2, 1e-24)
                s = jnp.maximum(q2 - (2.0 * c) * pn + (c * c) * n2, 0.0)
                plsc.store_scatter(
                    sbuf, [jnp.broadcast_to(g * CH + i, (L,))], s, mask=lane15)

        for k in range(RPW // L):
            sl = pl.ds(k * L, L)
            x = sbuf[sl]
            obuf[sl] = x * _rsqrt_nr(x)
        pltpu.sync_copy(obuf, out_hbm.at[pl.ds(base, RPW)])

    return vec2tail


_vec2tail = _make_kernel()


def kernel(h, r, t, ent_embedding, rel_hyperplane_embedding,
           rel_norm_embedding):
    rel = jnp.concatenate([rel_hyperplane_embedding, rel_norm_embedding],
                          axis=1)
    return _vec2tail(h.astype(jnp.int32), r.astype(jnp.int32),
                     t.astype(jnp.int32), ent_embedding, rel)


# speedup vs baseline: 4.0732x; 1.0230x over previous
import functools

import jax
import jax.numpy as jnp
from jax import lax
from jax.experimental import pallas as pl
from jax.experimental.pallas import tpu as pltpu
from jax.experimental.pallas import tpu_sc as plsc

D = 128
B = 16384
NC, NS, L = 2, 16, 16
NW = NC * NS
RPW = B // NW
CH = 64
NCHUNK = RPW // CH
NBUF = 3


def _rsqrt_nr(x):
    xi = plsc.bitcast(x, jnp.int32)
    yi = jnp.int32(0x5F3759DF) - (xi >> 1)
    y = plsc.bitcast(yi, jnp.float32)
    for _ in range(3):
        y = y * (1.5 - (0.5 * x) * y * y)
    return y


def _make_kernel():
    mesh = plsc.VectorSubcoreMesh(core_axis_name="c", subcore_axis_name="s")

    @functools.partial(
        pl.kernel,
        mesh=mesh,
        out_type=jax.ShapeDtypeStruct((B,), jnp.float32),
        compiler_params=pltpu.CompilerParams(needs_layout_passes=False),
        scratch_types=[
            pltpu.VMEM((RPW,), jnp.int32),
            pltpu.VMEM((RPW,), jnp.int32),
            pltpu.VMEM((RPW,), jnp.int32),
            pltpu.VMEM((NBUF, CH, D), jnp.float32),
            pltpu.VMEM((NBUF, CH, D), jnp.float32),
            pltpu.VMEM((NBUF, CH, D), jnp.float32),
            pltpu.VMEM((NBUF, CH, D), jnp.float32),
            pltpu.VMEM((RPW,), jnp.float32),
            pltpu.VMEM((RPW,), jnp.float32),
            pltpu.SemaphoreType.DMA,
            pltpu.SemaphoreType.DMA,
            pltpu.SemaphoreType.DMA,
            pltpu.SemaphoreType.DMA,
        ],
    )
    def vec2tail(h_hbm, r_hbm, t_hbm, ent_hbm, hyp_hbm, nrm_hbm, out_hbm,
                 hidx, tidx, ridx, hbuf, tbuf, dbuf, nbuf, sbuf, obuf,
                 sem0, sem1, sem2, isem):
        wid = lax.axis_index("s") * NC + lax.axis_index("c")
        base = wid * RPW
        icps = [
            pltpu.async_copy(h_hbm.at[pl.ds(base, RPW)], hidx, isem),
            pltpu.async_copy(t_hbm.at[pl.ds(base, RPW)], tidx, isem),
            pltpu.async_copy(r_hbm.at[pl.ds(base, RPW)], ridx, isem),
        ]
        for cp in icps:
            cp.wait()

        sems = (sem0, sem1, sem2)
        lane15 = lax.iota(jnp.int32, L) == (L - 1)

        def fire(g):
            slot = g % NBUF
            sem = sems[slot]
            sl = pl.ds(g * CH, CH)
            return [
                pltpu.async_copy(ent_hbm.at[hidx.at[sl]], hbuf.at[slot], sem),
                pltpu.async_copy(ent_hbm.at[tidx.at[sl]], tbuf.at[slot], sem),
                pltpu.async_copy(hyp_hbm.at[ridx.at[sl]], dbuf.at[slot], sem),
                pltpu.async_copy(nrm_hbm.at[ridx.at[sl]], nbuf.at[slot], sem),
            ]

        pending = {g: fire(g) for g in range(min(NBUF - 1, NCHUNK))}
        for g in range(NCHUNK):
            if g + NBUF - 1 < NCHUNK:
                pending[g + NBUF - 1] = fire(g + NBUF - 1)
            for cp in pending.pop(g):
                cp.wait()
            slot = g % NBUF

            @plsc.parallel_loop(0, CH, 1, unroll=4)
            def row(i, slot=slot, g=g):
                accq = accp = accn = accb = None
                for cb in range(D // L):
                    sl = pl.ds(cb * L, L)
                    hv = hbuf[slot, i, sl]
                    tv = tbuf[slot, i, sl]
                    dv = dbuf[slot, i, sl]
                    nv = nbuf[slot, i, sl]
                    u = hv - tv + dv
                    if cb == 0:
                        accq, accp = u * u, u * nv
                        accn, accb = nv * nv, dv * nv
                    else:
                        accq = accq + u * u
                        accp = accp + u * nv
                        accn = accn + nv * nv
                        accb = accb + dv * nv
                q2 = plsc.cumsum(accq)
                pn = plsc.cumsum(accp)
                n2 = plsc.cumsum(accn)
                bb = plsc.cumsum(accb)
                c = (pn - bb) / jnp.maximum(n2, 1e-24)
                s = jnp.maximum(q2 - (2.0 * c) * pn + (c * c) * n2, 0.0)
                plsc.store_scatter(
                    sbuf, [jnp.broadcast_to(g * CH + i, (L,))], s, mask=lane15)

        for k in range(RPW // L):
            sl = pl.ds(k * L, L)
            x = sbuf[sl]
            obuf[sl] = x * _rsqrt_nr(x)
        pltpu.sync_copy(obuf, out_hbm.at[pl.ds(base, RPW)])

    return vec2tail


_vec2tail = _make_kernel()


def kernel(h, r, t, ent_embedding, rel_hyperplane_embedding,
           rel_norm_embedding):
    return _vec2tail(h.astype(jnp.int32), r.astype(jnp.int32),
                     t.astype(jnp.int32), ent_embedding,
                     rel_hyperplane_embedding, rel_norm_embedding)
